# baseline (speedup 1.0000x reference)
import functools

import jax
import jax.numpy as jnp
from jax import lax
from jax.experimental import pallas as pl
from jax.experimental.pallas import tpu as pltpu
from jax.experimental.pallas import tpu_sc as plsc

N = 10000
E = 320000
D = 128

NC = 2
NS = 16
CHUNK = 128
E_TILE = E // NS
IB = 8
N_CH = 160
N_BLK = N_CH // IB
E_TILE_PAD = N_CH * CHUNK
ROWS_T = 632
N_PAD = ROWS_T * NS
BLK = N_PAD // 8


def _zero_rows(rows_v, nrow, ncol):
  def body(i, _):
    for k in range(ncol // 16):
      rows_v[i, pl.ds(k * 16, 16)] = jnp.zeros((16,), jnp.float32)
    return 0
  lax.fori_loop(0, nrow, body, 0)


@functools.cache
def _sc_kernels():
  mesh = plsc.VectorSubcoreMesh(
      core_axis_name="c", subcore_axis_name="s",
      num_cores=NC, num_subcores=NS)

  @functools.partial(
      pl.kernel,
      out_type=jax.ShapeDtypeStruct((NC * D, D), jnp.float32),
      mesh=mesh,
      scratch_types=[
          pltpu.VMEM((E_TILE_PAD,), jnp.int32),
          pltpu.VMEM((D, D), jnp.float32),
          pltpu.VMEM((D,), jnp.int32),
          pltpu.VMEM_SHARED((D, D), jnp.float32),
      ],
      compiler_params=pltpu.CompilerParams(needs_layout_passes=False),
  )
  def sc_degree(dst_hbm, zeros_hbm, deg_out, idx_v, hist_v, idr_v, acc):
    cid = lax.axis_index("c")
    sid = lax.axis_index("s")
    tid = cid * NS + sid
    pltpu.sync_copy(zeros_hbm, hist_v)
    for k in range(D // 16):
      idr_v[pl.ds(k * 16, 16)] = lax.iota(jnp.int32, 16) + (k * 16)
    pltpu.sync_copy(hist_v.at[pl.ds(0, 8)], acc.at[pl.ds(sid * 8, 8)])
    pltpu.sync_copy(dst_hbm.at[pl.ds(tid * E_TILE_PAD, E_TILE_PAD)], idx_v)
    plsc.subcore_barrier()

    ones16 = jnp.ones((16,), jnp.float32)
    def step(t, _):
      v = idx_v[pl.ds(t * 16, 16)]
      plsc.addupdate_scatter(
          hist_v, [lax.shift_right_logical(v, 7),
                   lax.bitwise_and(v, 127)], ones16)
      return 0
    lax.fori_loop(0, E_TILE_PAD // 16, step, 0)

    pltpu.sync_copy(hist_v, acc.at[idr_v], add=True)
    plsc.subcore_barrier()

    @pl.when(sid == 0)
    def _():
      pltpu.sync_copy(acc, deg_out.at[pl.ds(cid * D, D)])

  @functools.partial(
      pl.kernel,
      out_type=jax.ShapeDtypeStruct((NC * N_PAD, D), jnp.float32),
      mesh=mesh,
      scratch_types=[
          pltpu.VMEM((2, IB, CHUNK), jnp.int32),
          pltpu.VMEM((2, IB, CHUNK), jnp.int32),
          pltpu.VMEM((2, CHUNK, D), jnp.float32),
          pltpu.VMEM_SHARED((N_PAD, D), jnp.float32),
          pltpu.SemaphoreType.DMA,
          pltpu.SemaphoreType.DMA,
          pltpu.SemaphoreType.DMA,
      ],
  )
  def sc_scatter(g_hbm, src_hbm, dst_hbm, s_out, sidx_v, didx_v, rows_v, acc,
                 sem_g, sem_i, sem_s):
    cid = lax.axis_index("c")
    sid = lax.axis_index("s")
    tid = cid * NS + sid
    _zero_rows(rows_v.at[0], CHUNK, D)
    offs = [0, 64, 128, 192, 256, 320, 384, 448, 512, 576]
    szs = [64] * 9 + [56]
    pltpu.sync_copy(src_hbm.at[pl.ds(tid * N_CH, IB)], sidx_v.at[0])
    pltpu.sync_copy(dst_hbm.at[pl.ds(tid * N_CH, IB)], didx_v.at[0])
    plsc.subcore_barrier()

    def blk(b, _):
      bs = lax.rem(b, 2)
      base_ch = tid * N_CH + b * IB

      @pl.when(b > 0)
      def _():
        pltpu.make_async_copy(
            src_hbm.at[pl.ds(base_ch, IB)], sidx_v.at[bs], sem_i).wait()
        pltpu.make_async_copy(
            dst_hbm.at[pl.ds(base_ch, IB)], didx_v.at[bs], sem_i).wait()

      @pl.when(b + 1 < N_BLK)
      def _():
        pltpu.async_copy(src_hbm.at[pl.ds(base_ch + IB, IB)],
                         sidx_v.at[1 - bs], sem_i)
        pltpu.async_copy(dst_hbm.at[pl.ds(base_ch + IB, IB)],
                         didx_v.at[1 - bs], sem_i)

      pltpu.async_copy(g_hbm.at[sidx_v.at[bs, 0]], rows_v.at[0], sem_g)

      def step(p, _):
        ps = lax.rem(p, 2)
        pltpu.make_async_copy(
            g_hbm.at[pl.ds(0, CHUNK)], rows_v.at[ps], sem_g).wait()

        @pl.when(p + 1 < IB)
        def _():
          pltpu.async_copy(g_hbm.at[sidx_v.at[bs, p + 1]],
                           rows_v.at[lax.rem(p + 1, 2)], sem_g)
        return 0
      lax.fori_loop(0, IB, step, 0)
      return 0
    lax.fori_loop(0, N_BLK, blk, 0)

    plsc.subcore_barrier()
    base = cid * N_PAD + sid * ROWS_T
    for o, s in zip(offs, szs):
      pltpu.sync_copy(acc.at[pl.ds(sid * ROWS_T + o, s)],
                      s_out.at[pl.ds(base + o, s)])

  return sc_degree, sc_scatter


def _dinv_block(deg_blk, i):
  row = lax.broadcasted_iota(jnp.int32, (BLK, 1), 0) + i * BLK
  return jnp.where(row < N, lax.rsqrt(deg_blk + 1.0), 0.0)


def _mm_scale_body(x_ref, w_ref, deg_ref, o_ref):
  i = pl.program_id(1)
  dinv = _dinv_block(deg_ref[0], i)
  o_ref[0] = jnp.dot(x_ref[0], w_ref[...],
                     preferred_element_type=jnp.float32) * dinv


def _combine_mm_body(s_ref, g_ref, deg_ref, b_ref, w_ref, o_ref):
  i = pl.program_id(1)
  dinv = _dinv_block(deg_ref[0], i)
  h = jnp.maximum(dinv * (s_ref[0] + g_ref[0]) + b_ref[...], 0.0)
  o_ref[0] = jnp.dot(h, w_ref[...],
                     preferred_element_type=jnp.float32) * dinv


def _combine_stats_body(s_ref, g_ref, deg_ref, b_ref, o_ref, st_ref, acc):
  i = pl.program_id(1)
  dinv = _dinv_block(deg_ref[0], i)
  h = dinv * (s_ref[0] + g_ref[0]) + b_ref[...]
  o_ref[0] = h
  row = lax.broadcasted_iota(jnp.int32, (BLK, 1), 0) + i * BLK
  hm = jnp.where(row < N, h, 0.0)

  @pl.when(i == 0)
  def _():
    acc[...] = jnp.zeros((8, D), jnp.float32)

  acc[0, :] += jnp.sum(hm, axis=0)
  acc[1, :] += jnp.sum(hm * hm, axis=0)

  @pl.when(i == 7)
  def _():
    st_ref[0] = acc[...]


def _apply_body(h_ref, st_ref, z1_ref, z2_ref):
  for g, z_ref in ((0, z1_ref), (1, z2_ref)):
    s0 = st_ref[g, 0, :]
    s1 = st_ref[g, 1, :]
    mean = s0 / N
    var = (s1 - s0 * mean) / (N - 1)
    rstd = lax.rsqrt(var)
    z_ref[...] = (h_ref[g] - mean) * rstd


def _mm_scale(x, w, deg):
  return pl.pallas_call(
      _mm_scale_body,
      grid=(2, 8),
      in_specs=[
          pl.BlockSpec((1, BLK, D), lambda g, i: (g, i, 0)),
          pl.BlockSpec((D, D), lambda g, i: (0, 0)),
          pl.BlockSpec((1, BLK, 1), lambda g, i: (g, i, 0)),
      ],
      out_specs=pl.BlockSpec((1, BLK, D), lambda g, i: (g, i, 0)),
      out_shape=jax.ShapeDtypeStruct((2, N_PAD, D), jnp.float32),
  )(x, w, deg)


def _combine_mm(s, g, deg, b, w):
  return pl.pallas_call(
      _combine_mm_body,
      grid=(2, 8),
      in_specs=[
          pl.BlockSpec((1, BLK, D), lambda g_, i: (g_, i, 0)),
          pl.BlockSpec((1, BLK, D), lambda g_, i: (g_, i, 0)),
          pl.BlockSpec((1, BLK, 1), lambda g_, i: (g_, i, 0)),
          pl.BlockSpec((1, D), lambda g_, i: (0, 0)),
          pl.BlockSpec((D, D), lambda g_, i: (0, 0)),
      ],
      out_specs=pl.BlockSpec((1, BLK, D), lambda g_, i: (g_, i, 0)),
      out_shape=jax.ShapeDtypeStruct((2, N_PAD, D), jnp.float32),
  )(s, g, deg, b, w)


def _combine_stats(s, g, deg, b):
  return pl.pallas_call(
      _combine_stats_body,
      grid=(2, 8),
      in_specs=[
          pl.BlockSpec((1, BLK, D), lambda g_, i: (g_, i, 0)),
          pl.BlockSpec((1, BLK, D), lambda g_, i: (g_, i, 0)),
          pl.BlockSpec((1, BLK, 1), lambda g_, i: (g_, i, 0)),
          pl.BlockSpec((1, D), lambda g_, i: (0, 0)),
      ],
      out_specs=[
          pl.BlockSpec((1, BLK, D), lambda g_, i: (g_, i, 0)),
          pl.BlockSpec((1, 8, D), lambda g_, i: (g_, 0, 0)),
      ],
      out_shape=[
          jax.ShapeDtypeStruct((2, N_PAD, D), jnp.float32),
          jax.ShapeDtypeStruct((2, 8, D), jnp.float32),
      ],
      scratch_shapes=[pltpu.VMEM((8, D), jnp.float32)],
  )(s, g, deg, b)


def _standardize(h, st):
  return pl.pallas_call(
      _apply_body,
      grid=(8,),
      in_specs=[
          pl.BlockSpec((2, BLK, D), lambda i: (0, i, 0)),
          pl.BlockSpec((2, 8, D), lambda i: (0, 0, 0)),
      ],
      out_specs=[
          pl.BlockSpec((BLK, D), lambda i: (i, 0)),
          pl.BlockSpec((BLK, D), lambda i: (i, 0)),
      ],
      out_shape=[
          jax.ShapeDtypeStruct((N, D), jnp.float32),
          jax.ShapeDtypeStruct((N, D), jnp.float32),
      ],
  )(h, st)


def _prep_idx(col1, col2, off2):
  def one(col, off):
    a = col.reshape(NS, E_TILE) + off
    a = jnp.pad(a, ((0, 0), (0, E_TILE_PAD - E_TILE)),
                constant_values=off + N)
    return a.reshape(NS * N_CH, CHUNK)
  return jnp.concatenate([one(col1, 0), one(col2, off2)], axis=0)


def kernel(x1, edge_index1, x2, edge_index2, W0, b0, W1, b1):
  src = _prep_idx(edge_index1[0], edge_index2[0], N_PAD)
  dst = _prep_idx(edge_index1[1], edge_index2[1], 0)

  x = jnp.stack([x1, x2])
  x = jnp.pad(x, ((0, 0), (0, N_PAD - N), (0, 0)))

  sc_degree, sc_scatter = _sc_kernels()

  degh = sc_degree(dst.reshape(-1), jnp.zeros((D, D), jnp.float32))
  deg = degh.reshape(2, D * D)[:, :N_PAD, None]

  b0r = b0.reshape(1, D)
  b1r = b1.reshape(1, D)

  g0 = _mm_scale(x, W0, deg)
  s0 = sc_scatter(g0.reshape(NC * N_PAD, D), src, dst)
  g1 = _combine_mm(s0.reshape(2, N_PAD, D), g0, deg, b0r, W1)
  s1 = sc_scatter(g1.reshape(NC * N_PAD, D), src, dst)
  h2, st = _combine_stats(s1.reshape(2, N_PAD, D), g1, deg, b1r)
  z1, z2 = _standardize(h2, st)
  return (z1, z2)


# speedup vs baseline: 1.0562x; 1.0562x over previous
import functools

import jax
import jax.numpy as jnp
from jax import lax
from jax.experimental import pallas as pl
from jax.experimental.pallas import tpu as pltpu
from jax.experimental.pallas import tpu_sc as plsc

N = 10000
E = 320000
D = 128

NC = 2
NS = 16
CHUNK = 64
E_TILE = E // NS
IB = 32
N_CH = 320
N_BLK = N_CH // IB
E_TILE_PAD = N_CH * CHUNK
ROWS_T = 632
N_PAD = ROWS_T * NS
BLK = N_PAD // 8


def _zero_rows(rows_v, nrow, ncol):
  def body(i, _):
    for k in range(ncol // 16):
      rows_v[i, pl.ds(k * 16, 16)] = jnp.zeros((16,), jnp.float32)
    return 0
  lax.fori_loop(0, nrow, body, 0)


@functools.cache
def _sc_kernels():
  mesh = plsc.VectorSubcoreMesh(
      core_axis_name="c", subcore_axis_name="s",
      num_cores=NC, num_subcores=NS)

  @functools.partial(
      pl.kernel,
      out_type=jax.ShapeDtypeStruct((NC * D, D), jnp.float32),
      mesh=mesh,
      scratch_types=[
          pltpu.VMEM((E_TILE_PAD,), jnp.int32),
          pltpu.VMEM((D, D), jnp.float32),
          pltpu.VMEM((D,), jnp.int32),
          pltpu.VMEM_SHARED((D, D), jnp.float32),
      ],
      compiler_params=pltpu.CompilerParams(needs_layout_passes=False),
  )
  def sc_degree(dst_hbm, zeros_hbm, deg_out, idx_v, hist_v, idr_v, acc):
    cid = lax.axis_index("c")
    sid = lax.axis_index("s")
    tid = cid * NS + sid
    pltpu.sync_copy(zeros_hbm, hist_v)
    for k in range(D // 16):
      idr_v[pl.ds(k * 16, 16)] = lax.iota(jnp.int32, 16) + (k * 16)
    pltpu.sync_copy(hist_v.at[pl.ds(0, 8)], acc.at[pl.ds(sid * 8, 8)])
    pltpu.sync_copy(dst_hbm.at[pl.ds(tid * E_TILE_PAD, E_TILE_PAD)], idx_v)
    plsc.subcore_barrier()

    ones16 = jnp.ones((16,), jnp.float32)
    def step(t, _):
      v = idx_v[pl.ds(t * 16, 16)]
      plsc.addupdate_scatter(
          hist_v, [lax.shift_right_logical(v, 7),
                   lax.bitwise_and(v, 127)], ones16)
      return 0
    lax.fori_loop(0, E_TILE_PAD // 16, step, 0)

    pltpu.sync_copy(hist_v, acc.at[idr_v], add=True)
    plsc.subcore_barrier()

    @pl.when(sid == 0)
    def _():
      pltpu.sync_copy(acc, deg_out.at[pl.ds(cid * D, D)])

  @functools.partial(
      pl.kernel,
      out_type=jax.ShapeDtypeStruct((NC * N_PAD, D), jnp.float32),
      mesh=mesh,
      scratch_types=[
          pltpu.VMEM((2, IB, CHUNK), jnp.int32),
          pltpu.VMEM((2, IB, CHUNK), jnp.int32),
          pltpu.VMEM((4, CHUNK, D), jnp.float32),
          pltpu.VMEM_SHARED((N_PAD, D), jnp.float32),
          pltpu.SemaphoreType.DMA,
          pltpu.SemaphoreType.DMA,
          pltpu.SemaphoreType.DMA,
      ],
  )
  def sc_scatter(g_hbm, src_hbm, dst_hbm, s_out, sidx_v, didx_v, rows_v, acc,
                 sem_g, sem_i, sem_s):
    cid = lax.axis_index("c")
    sid = lax.axis_index("s")
    tid = cid * NS + sid
    _zero_rows(rows_v.at[0], CHUNK, D)
    offs = [0, 64, 128, 192, 256, 320, 384, 448, 512, 576]
    szs = [64] * 9 + [56]
    for o, s in zip(offs, szs):
      pltpu.sync_copy(rows_v.at[0, pl.ds(0, s)],
                      acc.at[pl.ds(sid * ROWS_T + o, s)])
    pltpu.sync_copy(src_hbm.at[pl.ds(tid * N_CH, IB)], sidx_v.at[0])
    pltpu.sync_copy(dst_hbm.at[pl.ds(tid * N_CH, IB)], didx_v.at[0])
    plsc.subcore_barrier()

    def blk(b, _):
      bs = lax.rem(b, 2)
      base_ch = tid * N_CH + b * IB

      @pl.when(b > 0)
      def _():
        pltpu.make_async_copy(
            src_hbm.at[pl.ds(base_ch, IB)], sidx_v.at[bs], sem_i).wait()
        pltpu.make_async_copy(
            dst_hbm.at[pl.ds(base_ch, IB)], didx_v.at[bs], sem_i).wait()

      @pl.when(b + 1 < N_BLK)
      def _():
        pltpu.async_copy(src_hbm.at[pl.ds(base_ch + IB, IB)],
                         sidx_v.at[1 - bs], sem_i)
        pltpu.async_copy(dst_hbm.at[pl.ds(base_ch + IB, IB)],
                         didx_v.at[1 - bs], sem_i)

      pltpu.async_copy(g_hbm.at[sidx_v.at[bs, 0]], rows_v.at[0], sem_g)
      pltpu.async_copy(g_hbm.at[sidx_v.at[bs, 1]], rows_v.at[1], sem_g)
      pltpu.async_copy(g_hbm.at[sidx_v.at[bs, 2]], rows_v.at[2], sem_g)

      def step(p, _):
        ps = lax.rem(p, 4)
        pltpu.make_async_copy(
            g_hbm.at[pl.ds(0, CHUNK)], rows_v.at[ps], sem_g).wait()
        pltpu.sync_copy(rows_v.at[ps], acc.at[didx_v.at[bs, p]], add=True)

        @pl.when(p + 3 < IB)
        def _():
          pltpu.async_copy(g_hbm.at[sidx_v.at[bs, p + 3]],
                           rows_v.at[lax.rem(p + 3, 4)], sem_g)
        return 0
      lax.fori_loop(0, IB, step, 0)
      return 0
    lax.fori_loop(0, N_BLK, blk, 0)

    plsc.subcore_barrier()
    base = cid * N_PAD + sid * ROWS_T
    for o, s in zip(offs, szs):
      pltpu.sync_copy(acc.at[pl.ds(sid * ROWS_T + o, s)],
                      s_out.at[pl.ds(base + o, s)])

  return sc_degree, sc_scatter


def _dinv_block(deg_blk, i):
  row = lax.broadcasted_iota(jnp.int32, (BLK, 1), 0) + i * BLK
  return jnp.where(row < N, lax.rsqrt(deg_blk + 1.0), 0.0)


def _mm_scale_body(x_ref, w_ref, deg_ref, o_ref):
  i = pl.program_id(1)
  dinv = _dinv_block(deg_ref[0], i)
  o_ref[0] = jnp.dot(x_ref[0], w_ref[...],
                     preferred_element_type=jnp.float32) * dinv


def _combine_mm_body(s_ref, g_ref, deg_ref, b_ref, w_ref, o_ref):
  i = pl.program_id(1)
  dinv = _dinv_block(deg_ref[0], i)
  h = jnp.maximum(dinv * (s_ref[0] + g_ref[0]) + b_ref[...], 0.0)
  o_ref[0] = jnp.dot(h, w_ref[...],
                     preferred_element_type=jnp.float32) * dinv


def _combine_stats_body(s_ref, g_ref, deg_ref, b_ref, o_ref, st_ref, acc):
  i = pl.program_id(1)
  dinv = _dinv_block(deg_ref[0], i)
  h = dinv * (s_ref[0] + g_ref[0]) + b_ref[...]
  o_ref[0] = h
  row = lax.broadcasted_iota(jnp.int32, (BLK, 1), 0) + i * BLK
  hm = jnp.where(row < N, h, 0.0)

  @pl.when(i == 0)
  def _():
    acc[...] = jnp.zeros((8, D), jnp.float32)

  acc[0, :] += jnp.sum(hm, axis=0)
  acc[1, :] += jnp.sum(hm * hm, axis=0)

  @pl.when(i == 7)
  def _():
    st_ref[0] = acc[...]


def _apply_body(h_ref, st_ref, z1_ref, z2_ref):
  for g, z_ref in ((0, z1_ref), (1, z2_ref)):
    s0 = st_ref[g, 0, :]
    s1 = st_ref[g, 1, :]
    mean = s0 / N
    var = (s1 - s0 * mean) / (N - 1)
    rstd = lax.rsqrt(var)
    z_ref[...] = (h_ref[g] - mean) * rstd


def _mm_scale(x, w, deg):
  return pl.pallas_call(
      _mm_scale_body,
      grid=(2, 8),
      in_specs=[
          pl.BlockSpec((1, BLK, D), lambda g, i: (g, i, 0)),
          pl.BlockSpec((D, D), lambda g, i: (0, 0)),
          pl.BlockSpec((1, BLK, 1), lambda g, i: (g, i, 0)),
      ],
      out_specs=pl.BlockSpec((1, BLK, D), lambda g, i: (g, i, 0)),
      out_shape=jax.ShapeDtypeStruct((2, N_PAD, D), jnp.float32),
  )(x, w, deg)


def _combine_mm(s, g, deg, b, w):
  return pl.pallas_call(
      _combine_mm_body,
      grid=(2, 8),
      in_specs=[
          pl.BlockSpec((1, BLK, D), lambda g_, i: (g_, i, 0)),
          pl.BlockSpec((1, BLK, D), lambda g_, i: (g_, i, 0)),
          pl.BlockSpec((1, BLK, 1), lambda g_, i: (g_, i, 0)),
          pl.BlockSpec((1, D), lambda g_, i: (0, 0)),
          pl.BlockSpec((D, D), lambda g_, i: (0, 0)),
      ],
      out_specs=pl.BlockSpec((1, BLK, D), lambda g_, i: (g_, i, 0)),
      out_shape=jax.ShapeDtypeStruct((2, N_PAD, D), jnp.float32),
  )(s, g, deg, b, w)


def _combine_stats(s, g, deg, b):
  return pl.pallas_call(
      _combine_stats_body,
      grid=(2, 8),
      in_specs=[
          pl.BlockSpec((1, BLK, D), lambda g_, i: (g_, i, 0)),
          pl.BlockSpec((1, BLK, D), lambda g_, i: (g_, i, 0)),
          pl.BlockSpec((1, BLK, 1), lambda g_, i: (g_, i, 0)),
          pl.BlockSpec((1, D), lambda g_, i: (0, 0)),
      ],
      out_specs=[
          pl.BlockSpec((1, BLK, D), lambda g_, i: (g_, i, 0)),
          pl.BlockSpec((1, 8, D), lambda g_, i: (g_, 0, 0)),
      ],
      out_shape=[
          jax.ShapeDtypeStruct((2, N_PAD, D), jnp.float32),
          jax.ShapeDtypeStruct((2, 8, D), jnp.float32),
      ],
      scratch_shapes=[pltpu.VMEM((8, D), jnp.float32)],
  )(s, g, deg, b)


def _standardize(h, st):
  return pl.pallas_call(
      _apply_body,
      grid=(8,),
      in_specs=[
          pl.BlockSpec((2, BLK, D), lambda i: (0, i, 0)),
          pl.BlockSpec((2, 8, D), lambda i: (0, 0, 0)),
      ],
      out_specs=[
          pl.BlockSpec((BLK, D), lambda i: (i, 0)),
          pl.BlockSpec((BLK, D), lambda i: (i, 0)),
      ],
      out_shape=[
          jax.ShapeDtypeStruct((N, D), jnp.float32),
          jax.ShapeDtypeStruct((N, D), jnp.float32),
      ],
  )(h, st)


def _prep_idx(col1, col2, off2):
  def one(col, off):
    a = col.reshape(NS, E_TILE) + off
    a = jnp.pad(a, ((0, 0), (0, E_TILE_PAD - E_TILE)),
                constant_values=off + N)
    return a.reshape(NS * N_CH, CHUNK)
  return jnp.concatenate([one(col1, 0), one(col2, off2)], axis=0)


def kernel(x1, edge_index1, x2, edge_index2, W0, b0, W1, b1):
  src = _prep_idx(edge_index1[0], edge_index2[0], N_PAD)
  dst = _prep_idx(edge_index1[1], edge_index2[1], 0)

  x = jnp.stack([x1, x2])
  x = jnp.pad(x, ((0, 0), (0, N_PAD - N), (0, 0)))

  sc_degree, sc_scatter = _sc_kernels()

  degh = sc_degree(dst.reshape(-1), jnp.zeros((D, D), jnp.float32))
  deg = degh.reshape(2, D * D)[:, :N_PAD, None]

  b0r = b0.reshape(1, D)
  b1r = b1.reshape(1, D)

  g0 = _mm_scale(x, W0, deg)
  s0 = sc_scatter(g0.reshape(NC * N_PAD, D), src, dst)
  g1 = _combine_mm(s0.reshape(2, N_PAD, D), g0, deg, b0r, W1)
  s1 = sc_scatter(g1.reshape(NC * N_PAD, D), src, dst)
  h2, st = _combine_stats(s1.reshape(2, N_PAD, D), g1, deg, b1r)
  z1, z2 = _standardize(h2, st)
  return (z1, z2)
